# baseline (speedup 1.0000x reference)
import functools

import jax
import jax.numpy as jnp
from jax import lax
from jax.experimental import pallas as pl
from jax.experimental.pallas import tpu as pltpu
from jax.experimental.pallas import tpu_sc as plsc

B, C, H, W = 4, 96, 384, 384
CP = 128
HW = H * W
NPB = 20000
NPTS = B * NPB
NW = 32
PTS_MAIN = 2560
PTS_LAST = NPTS - (NW - 1) * PTS_MAIN
S = 128
CH_MAIN = PTS_MAIN // S
CH_LAST = PTS_LAST // S
L = 16

HT = 64
NHB = H // HT


def _tr_body(f_ref, o_ref):
    for t in range(HT):
        xi = lax.bitcast_convert_type(f_ref[0, :, t, :].T, jnp.int32)
        xr = xi + jnp.int32(0x8000)
        nxt = pltpu.roll(xr, W - 1, 0)
        packed = jnp.bitwise_or(lax.shift_right_logical(xr, 16),
                                jnp.bitwise_and(nxt, jnp.int32(-65536)))
        o_ref[0, t, :, 0:C] = packed


def _transpose(feat):
    return pl.pallas_call(
        _tr_body,
        grid=(B, NHB),
        in_specs=[pl.BlockSpec((1, C, HT, W), lambda b, h: (b, 0, h, 0))],
        out_specs=pl.BlockSpec((1, HT, W, CP), lambda b, h: (b, h, 0, 0)),
        out_shape=jax.ShapeDtypeStruct((B, H, W, CP), jnp.int32),
    )(feat)


def _splat(vec, l):
    idx = jnp.broadcast_to(l, (L,)).astype(jnp.int32)[:, None]
    dn = lax.GatherDimensionNumbers(
        offset_dims=(), collapsed_slice_dims=(0,), start_index_map=(0,))
    return lax.gather(vec, idx, dn, (1,),
                      mode=lax.GatherScatterMode.PROMISE_IN_BOUNDS)


@functools.cache
def _build_sc_sample():
    mesh = plsc.VectorSubcoreMesh(core_axis_name="c", subcore_axis_name="s",
                                  num_cores=2, num_subcores=16)
    return functools.partial(
        pl.kernel,
        out_type=jax.ShapeDtypeStruct((NPTS, CP), jnp.float32),
        mesh=mesh,
        scratch_types=[
            pltpu.VMEM((PTS_MAIN * 2,), jnp.float32),
            pltpu.VMEM((2, 2, S), jnp.int32),
            pltpu.VMEM((2, 4, S), jnp.float32),
            pltpu.VMEM((2, 2, S, CP), jnp.int32),
            pltpu.VMEM((2, S, CP), jnp.float32),
            pltpu.SemaphoreType.DMA,
            pltpu.SemaphoreType.DMA,
            pltpu.SemaphoreType.DMA,
            pltpu.SemaphoreType.DMA,
        ],
        compiler_params=pltpu.CompilerParams(needs_layout_passes=False,
                                             use_tc_tiling_on_sc=True),
    )(_sc_sample_body)


def _sc_sample_body(feat_hbm, pts_hbm, out_hbm, pts_v, idx_v, w_v, rows_v, out_v,
                    sem_g0, sem_g1, sem_o0, sem_o1):
    wid = lax.axis_index("s") * 2 + lax.axis_index("c")
    base = wid * PTS_MAIN
    nchunks = jnp.where(wid == NW - 1, CH_LAST, CH_MAIN)
    sem_g = (sem_g0, sem_g1)
    sem_o = (sem_o0, sem_o1)

    @pl.when(wid < NW - 1)
    def _():
        pltpu.sync_copy(pts_hbm.at[pl.ds(base * 2, PTS_MAIN * 2)], pts_v)

    @pl.when(wid == NW - 1)
    def _():
        pltpu.sync_copy(pts_hbm.at[pl.ds(base * 2, PTS_LAST * 2)],
                        pts_v.at[pl.ds(0, PTS_LAST * 2)])

    def stage_a(ci, bi):
        def grp_a(g, c2):
            lane = lax.iota(jnp.int32, L)
            p_loc = ci * S + g * L + lane
            pos = p_loc * 2
            px = plsc.load_gather(pts_v, [pos])
            py = plsc.load_gather(pts_v, [pos + 1])
            fx = (px + 1.0) * (0.5 * (W - 1))
            fy = (py + 1.0) * (0.5 * (H - 1))
            fx = jnp.minimum(jnp.maximum(fx, 0.0), float(W - 1))
            fy = jnp.minimum(jnp.maximum(fy, 0.0), float(H - 1))
            x0 = jnp.minimum(fx.astype(jnp.int32), W - 2)
            y0 = jnp.minimum(fy.astype(jnp.int32), H - 2)
            ax = fx - x0.astype(jnp.float32)
            ay = fy - y0.astype(jnp.float32)
            bidx = (base + p_loc) // NPB
            row = bidx * HW + y0 * W + x0
            sl = pl.ds(g * L, L)
            idx_v[bi, 0, sl] = row
            idx_v[bi, 1, sl] = row + W
            w_v[bi, 0, sl] = 1.0 - ax
            w_v[bi, 1, sl] = ax
            w_v[bi, 2, sl] = 1.0 - ay
            w_v[bi, 3, sl] = ay
            return c2

        lax.fori_loop(0, S // L, grp_a, 0)
        for k in range(2):
            pltpu.async_copy(feat_hbm.at[idx_v.at[bi, k]], rows_v.at[bi, k],
                             sem_g[bi])

    def drain_g(bi):
        for k in range(2):
            pltpu.make_async_copy(feat_hbm.at[idx_v.at[bi, k]],
                                  rows_v.at[bi, k], sem_g[bi]).wait()

    def drain_o(bi):
        pltpu.make_async_copy(out_v.at[bi], out_hbm.at[pl.ds(0, S)],
                              sem_o[bi]).wait()

    def stage_c(ci, bi):
        @pl.when(ci >= 2)
        def _():
            drain_o(bi)

        def grp_c(g, c2):
            wv = [w_v[bi, k, pl.ds(g * L, L)] for k in range(4)]

            for l in range(L):
                p = g * L + l
                wx0, wx1, wy0, wy1 = (_splat(wv[k], l) for k in range(4))
                for j in range(C // L):
                    sl = pl.ds(j * L, L)
                    v0 = rows_v[bi, 0, p, sl]
                    v1 = rows_v[bi, 1, p, sl]
                    a0 = plsc.bitcast(lax.shift_left(v0, 16), jnp.float32)
                    b0 = plsc.bitcast(
                        jnp.bitwise_and(v0, jnp.int32(-65536)), jnp.float32)
                    a1 = plsc.bitcast(lax.shift_left(v1, 16), jnp.float32)
                    b1 = plsc.bitcast(
                        jnp.bitwise_and(v1, jnp.int32(-65536)), jnp.float32)
                    acc = (a0 * wx0 + b0 * wx1) * wy0
                    acc = acc + (a1 * wx0 + b1 * wx1) * wy1
                    out_v[bi, p, sl] = acc
            return c2

        lax.fori_loop(0, S // L, grp_c, 0)
        pltpu.async_copy(out_v.at[bi], out_hbm.at[pl.ds(base + ci * S, S)],
                         sem_o[bi])

    stage_a(0, 0)

    def pair_body(pi, carry):
        c = 2 * pi
        stage_a(c + 1, 1)
        drain_g(0)
        stage_c(c, 0)

        @pl.when(c + 2 < nchunks)
        def _():
            stage_a(c + 2, 0)

        drain_g(1)
        stage_c(c + 1, 1)
        return carry

    lax.fori_loop(0, nchunks // 2, pair_body, 0)

    @pl.when(nchunks % 2 == 1)
    def _():
        drain_g(0)
        stage_c(nchunks - 1, 0)

    drain_o(0)
    drain_o(1)


def kernel(features, points):
    feat_t = _transpose(features).reshape(B * HW, CP)
    pts_flat = points.reshape(NPTS * 2)
    out = _build_sc_sample()(feat_t, pts_flat)
    return out[:, :C].reshape(B, NPB, C)


# speedup vs baseline: 1.0019x; 1.0019x over previous
import functools

import jax
import jax.numpy as jnp
from jax import lax
from jax.experimental import pallas as pl
from jax.experimental.pallas import tpu as pltpu
from jax.experimental.pallas import tpu_sc as plsc

B, C, H, W = 4, 96, 384, 384
CP = 128
HW = H * W
NPB = 20000
NPTS = B * NPB
NW = 32
PTS_MAIN = 2560
PTS_LAST = NPTS - (NW - 1) * PTS_MAIN
S = 128
CH_MAIN = PTS_MAIN // S
CH_LAST = PTS_LAST // S
L = 16

HT = 64
NHB = H // HT


def _tr_body(f_ref, o_ref):
    for t in range(HT):
        xi = lax.bitcast_convert_type(f_ref[0, :, t, :].T, jnp.int32)
        xr = xi + jnp.int32(0x8000)
        nxt = pltpu.roll(xr, W - 1, 0)
        packed = jnp.bitwise_or(lax.shift_right_logical(xr, 16),
                                jnp.bitwise_and(nxt, jnp.int32(-65536)))
        o_ref[0, t, :, 0:C] = packed


def _transpose(feat):
    return pl.pallas_call(
        _tr_body,
        grid=(B, NHB),
        in_specs=[pl.BlockSpec((1, C, HT, W), lambda b, h: (b, 0, h, 0))],
        out_specs=pl.BlockSpec((1, HT, W, CP), lambda b, h: (b, h, 0, 0)),
        out_shape=jax.ShapeDtypeStruct((B, H, W, CP), jnp.int32),
    )(feat)


def _splat(vec, l):
    idx = jnp.broadcast_to(l, (L,)).astype(jnp.int32)[:, None]
    dn = lax.GatherDimensionNumbers(
        offset_dims=(), collapsed_slice_dims=(0,), start_index_map=(0,))
    return lax.gather(vec, idx, dn, (1,),
                      mode=lax.GatherScatterMode.PROMISE_IN_BOUNDS)


@functools.cache
def _build_sc_sample():
    mesh = plsc.VectorSubcoreMesh(core_axis_name="c", subcore_axis_name="s",
                                  num_cores=2, num_subcores=16)
    return functools.partial(
        pl.kernel,
        out_type=jax.ShapeDtypeStruct((NPTS, C), jnp.float32),
        mesh=mesh,
        scratch_types=[
            pltpu.VMEM((PTS_MAIN * 2,), jnp.float32),
            pltpu.VMEM((2, 2, S), jnp.int32),
            pltpu.VMEM((2, 4, S), jnp.float32),
            pltpu.VMEM((2, 2, S, CP), jnp.int32),
            pltpu.VMEM((2, S, C), jnp.float32),
            pltpu.SemaphoreType.DMA,
            pltpu.SemaphoreType.DMA,
            pltpu.SemaphoreType.DMA,
            pltpu.SemaphoreType.DMA,
        ],
        compiler_params=pltpu.CompilerParams(needs_layout_passes=False,
                                             use_tc_tiling_on_sc=True),
    )(_sc_sample_body)


def _sc_sample_body(feat_hbm, pts_hbm, out_hbm, pts_v, idx_v, w_v, rows_v, out_v,
                    sem_g0, sem_g1, sem_o0, sem_o1):
    wid = lax.axis_index("s") * 2 + lax.axis_index("c")
    base = wid * PTS_MAIN
    nchunks = jnp.where(wid == NW - 1, CH_LAST, CH_MAIN)
    sem_g = (sem_g0, sem_g1)
    sem_o = (sem_o0, sem_o1)

    @pl.when(wid < NW - 1)
    def _():
        pltpu.sync_copy(pts_hbm.at[pl.ds(base * 2, PTS_MAIN * 2)], pts_v)

    @pl.when(wid == NW - 1)
    def _():
        pltpu.sync_copy(pts_hbm.at[pl.ds(base * 2, PTS_LAST * 2)],
                        pts_v.at[pl.ds(0, PTS_LAST * 2)])

    def stage_a(ci, bi):
        def grp_a(g, c2):
            lane = lax.iota(jnp.int32, L)
            p_loc = ci * S + g * L + lane
            pos = p_loc * 2
            px = plsc.load_gather(pts_v, [pos])
            py = plsc.load_gather(pts_v, [pos + 1])
            fx = (px + 1.0) * (0.5 * (W - 1))
            fy = (py + 1.0) * (0.5 * (H - 1))
            fx = jnp.minimum(jnp.maximum(fx, 0.0), float(W - 1))
            fy = jnp.minimum(jnp.maximum(fy, 0.0), float(H - 1))
            x0 = jnp.minimum(fx.astype(jnp.int32), W - 2)
            y0 = jnp.minimum(fy.astype(jnp.int32), H - 2)
            ax = fx - x0.astype(jnp.float32)
            ay = fy - y0.astype(jnp.float32)
            bidx = (base + p_loc) // NPB
            row = bidx * HW + y0 * W + x0
            sl = pl.ds(g * L, L)
            idx_v[bi, 0, sl] = row
            idx_v[bi, 1, sl] = row + W
            w_v[bi, 0, sl] = 1.0 - ax
            w_v[bi, 1, sl] = ax
            w_v[bi, 2, sl] = 1.0 - ay
            w_v[bi, 3, sl] = ay
            return c2

        lax.fori_loop(0, S // L, grp_a, 0)
        for k in range(2):
            pltpu.async_copy(feat_hbm.at[idx_v.at[bi, k]], rows_v.at[bi, k],
                             sem_g[bi])

    def drain_g(bi):
        for k in range(2):
            pltpu.make_async_copy(feat_hbm.at[idx_v.at[bi, k]],
                                  rows_v.at[bi, k], sem_g[bi]).wait()

    def drain_o(bi):
        pltpu.make_async_copy(out_v.at[bi], out_hbm.at[pl.ds(0, S)],
                              sem_o[bi]).wait()

    def stage_c(ci, bi):
        @pl.when(ci >= 2)
        def _():
            drain_o(bi)

        def grp_c(g, c2):
            wv = [w_v[bi, k, pl.ds(g * L, L)] for k in range(4)]

            for l in range(L):
                p = g * L + l
                wx0, wx1, wy0, wy1 = (_splat(wv[k], l) for k in range(4))
                for j in range(C // L):
                    sl = pl.ds(j * L, L)
                    v0 = rows_v[bi, 0, p, sl]
                    v1 = rows_v[bi, 1, p, sl]
                    a0 = plsc.bitcast(lax.shift_left(v0, 16), jnp.float32)
                    b0 = plsc.bitcast(
                        jnp.bitwise_and(v0, jnp.int32(-65536)), jnp.float32)
                    a1 = plsc.bitcast(lax.shift_left(v1, 16), jnp.float32)
                    b1 = plsc.bitcast(
                        jnp.bitwise_and(v1, jnp.int32(-65536)), jnp.float32)
                    acc = (a0 * wx0 + b0 * wx1) * wy0
                    acc = acc + (a1 * wx0 + b1 * wx1) * wy1
                    out_v[bi, p, sl] = acc
            return c2

        lax.fori_loop(0, S // L, grp_c, 0)
        pltpu.async_copy(out_v.at[bi], out_hbm.at[pl.ds(base + ci * S, S)],
                         sem_o[bi])

    stage_a(0, 0)

    def pair_body(pi, carry):
        c = 2 * pi
        stage_a(c + 1, 1)
        drain_g(0)
        stage_c(c, 0)

        @pl.when(c + 2 < nchunks)
        def _():
            stage_a(c + 2, 0)

        drain_g(1)
        stage_c(c + 1, 1)
        return carry

    lax.fori_loop(0, nchunks // 2, pair_body, 0)

    @pl.when(nchunks % 2 == 1)
    def _():
        drain_g(0)
        stage_c(nchunks - 1, 0)

    drain_o(0)
    drain_o(1)


def kernel(features, points):
    feat_t = _transpose(features).reshape(B * HW, CP)
    pts_flat = points.reshape(NPTS * 2)
    out = _build_sc_sample()(feat_t, pts_flat)
    return out.reshape(B, NPB, C)
